# Initial kernel scaffold; baseline (speedup 1.0000x reference)
import functools

import jax
import jax.numpy as jnp
from jax import lax
from jax.experimental import pallas as pl
from jax.experimental.pallas import tpu as pltpu
from jax.experimental.pallas import tpu_sc as plsc

IDXW = 128
K = 4


def _scale_body(t_ref, o_ref):
    o_ref[...] = t_ref[...] * 8.0


def _scale_table(table):
    v, d = table.shape
    block_rows = 1000
    return pl.pallas_call(
        _scale_body,
        grid=(v // block_rows,),
        in_specs=[pl.BlockSpec((block_rows, d), lambda i: (i, 0))],
        out_specs=pl.BlockSpec((block_rows, d), lambda i: (i, 0)),
        out_shape=jax.ShapeDtypeStruct((v, d), table.dtype),
    )(table)


@functools.lru_cache(maxsize=None)
def _make_gather(nrows, d):
    info = plsc.get_sparse_core_info()
    nc, ns = info.num_cores, info.num_subcores
    nw = nc * ns
    rows_per_w = nrows // nw
    steps = rows_per_w // K
    mesh = plsc.VectorSubcoreMesh(core_axis_name="c", subcore_axis_name="s")

    @functools.partial(
        pl.kernel,
        mesh=mesh,
        out_type=jax.ShapeDtypeStruct((nrows, IDXW, d), jnp.float32),
        scratch_types=[
            pltpu.VMEM((K, IDXW), jnp.int32),
            pltpu.VMEM((K, IDXW, d), jnp.float32),
            pltpu.SemaphoreType.DMA,
        ],
    )
    def k(idx_hbm, table_hbm, out_hbm, idx_v, rows_v, sem):
        wid = lax.axis_index("s") * nc + lax.axis_index("c")
        base = wid * rows_per_w

        def body(g, carry):
            row = base + g * K
            pltpu.sync_copy(idx_hbm.at[pl.ds(row, K)], idx_v)
            copies = [
                pltpu.async_copy(table_hbm.at[idx_v.at[j]], rows_v.at[j], sem)
                for j in range(K)
            ]
            for c in copies:
                c.wait()
            pltpu.sync_copy(rows_v, out_hbm.at[pl.ds(row, K)])
            return carry

        lax.fori_loop(0, steps, body, 0)

    return k


def kernel(x, table):
    s1, s2 = x.shape
    v, d = table.shape
    b = s1 * s2
    assert b % IDXW == 0
    nrows = b // IDXW
    scaled = _scale_table(table)
    idx2 = jnp.reshape(x.astype(jnp.int32), (nrows, IDXW))
    out = _make_gather(nrows, d)(idx2, scaled)
    return jnp.reshape(out, (s1, s2, d))


# speedup vs baseline: 3.5187x; 3.5187x over previous
import functools

import jax
import jax.numpy as jnp
from jax import lax
from jax.experimental import pallas as pl
from jax.experimental.pallas import tpu as pltpu
from jax.experimental.pallas import tpu_sc as plsc

IDXW = 128
K = 4


def _scale_body(t_ref, o_ref):
    o_ref[...] = t_ref[...] * 8.0


def _scale_table(table):
    v, d = table.shape
    block_rows = 1000
    return pl.pallas_call(
        _scale_body,
        grid=(v // block_rows,),
        in_specs=[pl.BlockSpec((block_rows, d), lambda i: (i, 0))],
        out_specs=pl.BlockSpec((block_rows, d), lambda i: (i, 0)),
        out_shape=jax.ShapeDtypeStruct((v, d), table.dtype),
    )(table)


@functools.lru_cache(maxsize=None)
def _make_gather(nrows, d):
    info = plsc.get_sparse_core_info()
    nc, ns = info.num_cores, info.num_subcores
    nw = nc * ns
    rows_per_w = nrows // nw
    steps = rows_per_w // K
    mesh = plsc.VectorSubcoreMesh(core_axis_name="c", subcore_axis_name="s")

    @functools.partial(
        pl.kernel,
        mesh=mesh,
        out_type=jax.ShapeDtypeStruct((nrows, IDXW, d), jnp.float32),
        scratch_types=[
            pltpu.VMEM((K, IDXW), jnp.int32),
            pltpu.VMEM((K, IDXW, d), jnp.float32),
            pltpu.SemaphoreType.DMA,
        ],
        compiler_params=pltpu.CompilerParams(use_tc_tiling_on_sc=False),
    )
    def k(idx_hbm, table_hbm, out_hbm, idx_v, rows_v, sem):
        wid = lax.axis_index("s") * nc + lax.axis_index("c")
        base = wid * rows_per_w

        def body(g, carry):
            row = base + g * K
            pltpu.sync_copy(idx_hbm.at[pl.ds(row, K)], idx_v)
            copies = [
                pltpu.async_copy(table_hbm.at[idx_v.at[j]], rows_v.at[j], sem)
                for j in range(K)
            ]
            for c in copies:
                c.wait()
            pltpu.sync_copy(rows_v, out_hbm.at[pl.ds(row, K)])
            return carry

        lax.fori_loop(0, steps, body, 0)

    return k


def kernel(x, table):
    s1, s2 = x.shape
    v, d = table.shape
    b = s1 * s2
    assert b % IDXW == 0
    nrows = b // IDXW
    scaled = _scale_table(table)
    idx2 = jnp.reshape(x.astype(jnp.int32), (nrows, IDXW))
    out = _make_gather(nrows, d)(idx2, scaled)
    return jnp.reshape(out, (s1, s2, d))


# speedup vs baseline: 3.7467x; 1.0648x over previous
import functools

import jax
import jax.numpy as jnp
from jax import lax
from jax.experimental import pallas as pl
from jax.experimental.pallas import tpu as pltpu
from jax.experimental.pallas import tpu_sc as plsc

IDXW = 128
K = 4


def _scale_body(t_ref, o_ref):
    o_ref[...] = t_ref[...] * 8.0


def _scale_table(table):
    v, d = table.shape
    block_rows = 1000
    return pl.pallas_call(
        _scale_body,
        grid=(v // block_rows,),
        in_specs=[pl.BlockSpec((block_rows, d), lambda i: (i, 0))],
        out_specs=pl.BlockSpec((block_rows, d), lambda i: (i, 0)),
        out_shape=jax.ShapeDtypeStruct((v, d), table.dtype),
    )(table)


@functools.lru_cache(maxsize=None)
def _make_gather(nrows, d):
    info = plsc.get_sparse_core_info()
    nc, ns = info.num_cores, info.num_subcores
    nw = nc * ns
    rows_per_w = nrows // nw
    steps = rows_per_w // K
    assert steps % 2 == 0 and steps >= 4
    mesh = plsc.VectorSubcoreMesh(core_axis_name="c", subcore_axis_name="s")

    @functools.partial(
        pl.kernel,
        mesh=mesh,
        out_type=jax.ShapeDtypeStruct((nrows, IDXW, d), jnp.float32),
        scratch_types=[
            pltpu.VMEM((2, K, IDXW), jnp.int32),
            pltpu.VMEM((2, K, IDXW, d), jnp.float32),
            pltpu.SemaphoreType.DMA,
            pltpu.SemaphoreType.DMA,
            pltpu.SemaphoreType.DMA,
            pltpu.SemaphoreType.DMA,
            pltpu.SemaphoreType.DMA,
        ],
        compiler_params=pltpu.CompilerParams(use_tc_tiling_on_sc=False),
    )
    def k(idx_hbm, table_hbm, out_hbm, idx_v, rows_v, gsem, isem0, isem1,
          osem0, osem1):
        isems = (isem0, isem1)
        osems = (osem0, osem1)
        wid = lax.axis_index("s") * nc + lax.axis_index("c")
        base = wid * rows_per_w

        def do_step(g, b, first):
            row = base + g * K
            pltpu.make_async_copy(
                idx_hbm.at[pl.ds(row, K)], idx_v.at[b], isems[b]).wait()
            if not first:
                pltpu.make_async_copy(
                    rows_v.at[b], out_hbm.at[pl.ds(row, K)], osems[b]).wait()
            copies = [
                pltpu.async_copy(
                    table_hbm.at[idx_v.at[b, j]], rows_v.at[b, j], gsem)
                for j in range(K)
            ]
            for c in copies:
                c.wait()
            pltpu.async_copy(rows_v.at[b], out_hbm.at[pl.ds(row, K)], osems[b])
            nxt = g + 2

            @pl.when(nxt < steps)
            def _():
                pltpu.async_copy(
                    idx_hbm.at[pl.ds(base + nxt * K, K)], idx_v.at[b],
                    isems[b])

        pltpu.async_copy(idx_hbm.at[pl.ds(base, K)], idx_v.at[0], isems[0])
        pltpu.async_copy(idx_hbm.at[pl.ds(base + K, K)], idx_v.at[1], isems[1])
        do_step(0, 0, first=True)
        do_step(1, 1, first=True)

        def body(t, carry):
            g0 = 2 + 2 * t
            do_step(g0, 0, first=False)
            do_step(g0 + 1, 1, first=False)
            return carry

        lax.fori_loop(0, (steps - 2) // 2, body, 0)

        for b in (0, 1):
            pltpu.make_async_copy(
                rows_v.at[b], out_hbm.at[pl.ds(base, K)], osems[b]).wait()

    return k


def kernel(x, table):
    s1, s2 = x.shape
    v, d = table.shape
    b = s1 * s2
    assert b % IDXW == 0
    nrows = b // IDXW
    scaled = _scale_table(table)
    idx2 = jnp.reshape(x.astype(jnp.int32), (nrows, IDXW))
    out = _make_gather(nrows, d)(idx2, scaled)
    return jnp.reshape(out, (s1, s2, d))
